# Initial kernel scaffold; baseline (speedup 1.0000x reference)
import jax
import jax.numpy as jnp
from jax.experimental import pallas as pl


def kernel(x_cat, x_num, tables, W1, b1, g1, bt1, W2, b2, g2, bt2, W3, b3, g3, bt3, W4, b4):
    raise NotImplementedError("write your pallas kernel here")


# speedup vs baseline: 7.1093x; 7.1093x over previous
import functools

import jax
import jax.numpy as jnp
from jax import lax
from jax.experimental import pallas as pl
from jax.experimental.pallas import tpu as pltpu
from jax.experimental.pallas import tpu_sc as plsc

_GATHER_WINDOW = 128
_TB = 1024
_PREC = lax.Precision.HIGHEST


def _gather_sc(tables_flat, idx):
    num_idx = idx.shape[0]
    d = tables_flat.shape[1]
    idx2 = idx.reshape(1, num_idx)
    mesh = plsc.VectorSubcoreMesh(core_axis_name="c", subcore_axis_name="s")

    @functools.partial(
        pl.kernel,
        out_type=jax.ShapeDtypeStruct((num_idx, d), tables_flat.dtype),
        mesh=mesh,
        compiler_params=pltpu.CompilerParams(use_tc_tiling_on_sc=False),
    )
    def gather_kernel(x_hbm, i_hbm, o_hbm):
        def body(i_vmem, o_vmem):
            pltpu.sync_copy(x_hbm.at[i_vmem.at[0]], o_vmem)

        pltpu.emit_pipeline(
            body,
            grid=(num_idx // _GATHER_WINDOW,),
            in_specs=[pl.BlockSpec((1, _GATHER_WINDOW), lambda i: (0, i))],
            out_specs=[pl.BlockSpec((_GATHER_WINDOW, d), lambda i: (i, 0))],
            core_axis_name=("c", "s"),
            dimension_semantics=(pltpu.PARALLEL,),
        )(i_hbm, o_hbm)

    return gather_kernel(tables_flat, idx2)


def _ln_gelu(h, g, bt):
    mu = jnp.mean(h, axis=-1, keepdims=True)
    var = jnp.mean((h - mu) ** 2, axis=-1, keepdims=True)
    h = (h - mu) / jnp.sqrt(var + 1e-5) * g + bt
    return h * 0.5 * (1.0 + lax.erf(h * (2.0 ** -0.5)))


def _mlp_body(emb_ref, xnum_ref, w1a_ref, w1b_ref, b1_ref, g1_ref, bt1_ref,
              w2_ref, b2_ref, g2_ref, bt2_ref, w3_ref, b3_ref, g3_ref, bt3_ref,
              w4_ref, b4_ref, out_ref):
    dot = functools.partial(jnp.dot, preferred_element_type=jnp.float32,
                            precision=_PREC)
    h = dot(emb_ref[...], w1a_ref[...]) + dot(xnum_ref[...], w1b_ref[...])
    h = _ln_gelu(h + b1_ref[...], g1_ref[...], bt1_ref[...])
    h = _ln_gelu(dot(h, w2_ref[...]) + b2_ref[...], g2_ref[...], bt2_ref[...])
    h = _ln_gelu(dot(h, w3_ref[...]) + b3_ref[...], g3_ref[...], bt3_ref[...])
    z = dot(h, w4_ref[...]) + b4_ref[...]
    out_ref[...] = jax.nn.sigmoid(z)


def _mlp_tc(emb, xnum, w1a, w1b, b1, g1, bt1, w2, b2, g2, bt2,
            w3, b3, g3, bt3, w4, b4):
    bsz = emb.shape[0]
    grid = (bsz // _TB,)

    def tile(r):
        return pl.BlockSpec((_TB, r.shape[1]), lambda i: (i, 0))

    def full(r):
        return pl.BlockSpec(r.shape, lambda i: (0, 0))

    return pl.pallas_call(
        _mlp_body,
        grid=grid,
        in_specs=[tile(emb), tile(xnum)] + [full(r) for r in (
            w1a, w1b, b1, g1, bt1, w2, b2, g2, bt2, w3, b3, g3, bt3, w4, b4)],
        out_specs=pl.BlockSpec((_TB, 1), lambda i: (i, 0)),
        out_shape=jax.ShapeDtypeStruct((bsz, 1), jnp.float32),
        compiler_params=pltpu.CompilerParams(
            dimension_semantics=("arbitrary",)),
    )(emb, xnum, w1a, w1b, b1, g1, bt1, w2, b2, g2, bt2,
      w3, b3, g3, bt3, w4, b4)


def kernel(x_cat, x_num, tables, W1, b1, g1, bt1, W2, b2, g2, bt2,
           W3, b3, g3, bt3, W4, b4):
    f, v, d = tables.shape
    b = x_cat.shape[0]
    tables_flat = tables.reshape(f * v, d)
    offsets = (jnp.arange(f, dtype=jnp.int32) * v)[None, :]
    idx = (x_cat + offsets).reshape(-1)
    emb = _gather_sc(tables_flat, idx).reshape(b, f * d)
    w1a, w1b = W1[: f * d], W1[f * d:]
    row = lambda x: x.reshape(1, -1)
    return _mlp_tc(emb, x_num, w1a, w1b, row(b1), row(g1), row(bt1),
                   W2, row(b2), row(g2), row(bt2),
                   W3, row(b3), row(g3), row(bt3), W4, row(b4))


# speedup vs baseline: 10.1407x; 1.4264x over previous
import functools

import jax
import jax.numpy as jnp
from jax import lax
from jax.experimental import pallas as pl
from jax.experimental.pallas import tpu as pltpu
from jax.experimental.pallas import tpu_sc as plsc

_GATHER_WINDOW = 128
_TB = 1024
_VC = 20000
_PREC = lax.Precision.HIGHEST


def _linearize_tables_tc(tablesT):
    f, d, v = tablesT.shape
    rows_per_blk = v * d // 128
    chunk = 4000
    bounds = list(range(0, v, chunk))

    def body(in_ref, out_ref, y_ref):
        x = in_ref[0]
        for c0 in bounds:
            clen = min(chunk, v - c0)
            y_ref[0:clen, :] = x[:, c0:c0 + clen].T
            parts = [y_ref[pl.Slice(j, clen // 4, 4), :] for j in range(4)]
            out_ref[c0 // 4:(c0 + clen) // 4, :] = jnp.concatenate(
                parts, axis=1)

    return pl.pallas_call(
        body,
        grid=(f,),
        in_specs=[pl.BlockSpec((1, d, v), lambda i: (i, 0, 0))],
        out_specs=pl.BlockSpec((rows_per_blk, 128), lambda i: (i, 0)),
        out_shape=jax.ShapeDtypeStruct((f * v * d // 128, 128), jnp.float32),
        scratch_shapes=[pltpu.VMEM((chunk, d), jnp.float32)],
        compiler_params=pltpu.CompilerParams(
            dimension_semantics=("arbitrary",),
            vmem_limit_bytes=130 * 1024 * 1024),
    )(tablesT)


def _gather_sc(tables_flat, idx):
    num_idx = idx.shape[0]
    d = tables_flat.shape[1]
    idx2 = idx.reshape(1, num_idx)
    mesh = plsc.VectorSubcoreMesh(core_axis_name="c", subcore_axis_name="s")

    @functools.partial(
        pl.kernel,
        out_type=jax.ShapeDtypeStruct((num_idx, d), tables_flat.dtype),
        mesh=mesh,
        compiler_params=pltpu.CompilerParams(use_tc_tiling_on_sc=False),
    )
    def gather_kernel(x_hbm, i_hbm, o_hbm):
        def body(i_vmem, o_vmem):
            pltpu.sync_copy(x_hbm.at[i_vmem.at[0]], o_vmem)

        pltpu.emit_pipeline(
            body,
            grid=(num_idx // _GATHER_WINDOW,),
            in_specs=[pl.BlockSpec((1, _GATHER_WINDOW), lambda i: (0, i))],
            out_specs=[pl.BlockSpec((_GATHER_WINDOW, d), lambda i: (i, 0))],
            core_axis_name=("c", "s"),
            dimension_semantics=(pltpu.PARALLEL,),
        )(i_hbm, o_hbm)

    return gather_kernel(tables_flat, idx2)


def _ln_gelu(h, g, bt):
    mu = jnp.mean(h, axis=-1, keepdims=True)
    var = jnp.mean((h - mu) ** 2, axis=-1, keepdims=True)
    h = (h - mu) / jnp.sqrt(var + 1e-5) * g + bt
    return h * 0.5 * (1.0 + lax.erf(h * (2.0 ** -0.5)))


def _mlp_body(emb_ref, xnum_ref, w1a_ref, w1b_ref, b1_ref, g1_ref, bt1_ref,
              w2_ref, b2_ref, g2_ref, bt2_ref, w3_ref, b3_ref, g3_ref, bt3_ref,
              w4_ref, b4_ref, out_ref):
    dot = functools.partial(jnp.dot, preferred_element_type=jnp.float32,
                            precision=_PREC)
    h = dot(emb_ref[...], w1a_ref[...]) + dot(xnum_ref[...], w1b_ref[...])
    h = _ln_gelu(h + b1_ref[...], g1_ref[...], bt1_ref[...])
    h = _ln_gelu(dot(h, w2_ref[...]) + b2_ref[...], g2_ref[...], bt2_ref[...])
    h = _ln_gelu(dot(h, w3_ref[...]) + b3_ref[...], g3_ref[...], bt3_ref[...])
    z = dot(h, w4_ref[...]) + b4_ref[...]
    out_ref[...] = jax.nn.sigmoid(z)


def _mlp_tc(emb, xnum, w1a, w1b, b1, g1, bt1, w2, b2, g2, bt2,
            w3, b3, g3, bt3, w4, b4):
    bsz = emb.shape[0]
    grid = (bsz // _TB,)

    def tile(r):
        return pl.BlockSpec((_TB, r.shape[1]), lambda i: (i, 0))

    def full(r):
        return pl.BlockSpec(r.shape, lambda i: (0, 0))

    return pl.pallas_call(
        _mlp_body,
        grid=grid,
        in_specs=[tile(emb), tile(xnum)] + [full(r) for r in (
            w1a, w1b, b1, g1, bt1, w2, b2, g2, bt2, w3, b3, g3, bt3, w4, b4)],
        out_specs=pl.BlockSpec((_TB, 1), lambda i: (i, 0)),
        out_shape=jax.ShapeDtypeStruct((bsz, 1), jnp.float32),
        compiler_params=pltpu.CompilerParams(
            dimension_semantics=("arbitrary",)),
    )(emb, xnum, w1a, w1b, b1, g1, bt1, w2, b2, g2, bt2,
      w3, b3, g3, bt3, w4, b4)


def kernel(x_cat, x_num, tables, W1, b1, g1, bt1, W2, b2, g2, bt2,
           W3, b3, g3, bt3, W4, b4):
    f, v, d = tables.shape
    b = x_cat.shape[0]
    tables_flat = _linearize_tables_tc(
        jnp.swapaxes(tables, 1, 2)).reshape(f * v, d)
    offsets = (jnp.arange(f, dtype=jnp.int32) * v)[None, :]
    idx = (x_cat + offsets).reshape(-1)
    emb = _gather_sc(tables_flat, idx).reshape(b, f * d)
    w1a, w1b = W1[: f * d], W1[f * d:]
    row = lambda x: x.reshape(1, -1)
    return _mlp_tc(emb, x_num, w1a, w1b, row(b1), row(g1), row(bt1),
                   W2, row(b2), row(g2), row(bt2),
                   W3, row(b3), row(g3), row(bt3), W4, row(b4))


# speedup vs baseline: 10.1488x; 1.0008x over previous
import functools

import jax
import jax.numpy as jnp
from jax import lax
from jax.experimental import pallas as pl
from jax.experimental.pallas import tpu as pltpu
from jax.experimental.pallas import tpu_sc as plsc

_GATHER_WINDOW = 128
_TB = 1024
_VC = 20000
_PREC = lax.Precision.HIGHEST


def _linearize_tables_tc(tablesT):
    f, d, v = tablesT.shape
    rows_per_blk = v * d // 128
    chunk = 4000
    bounds = list(range(0, v, chunk))

    def body(in_ref, out_ref, y_ref):
        x = in_ref[0]
        for c0 in bounds:
            clen = min(chunk, v - c0)
            y_ref[0:clen, :] = x[:, c0:c0 + clen].T
            parts = [y_ref[pl.Slice(j, clen // 4, 4), :] for j in range(4)]
            out_ref[c0 // 4:(c0 + clen) // 4, :] = jnp.concatenate(
                parts, axis=1)

    return pl.pallas_call(
        body,
        grid=(f,),
        in_specs=[pl.BlockSpec((1, d, v), lambda i: (i, 0, 0))],
        out_specs=pl.BlockSpec((rows_per_blk, 128), lambda i: (i, 0)),
        out_shape=jax.ShapeDtypeStruct((f * v * d // 128, 128), jnp.float32),
        scratch_shapes=[pltpu.VMEM((chunk, d), jnp.float32)],
        compiler_params=pltpu.CompilerParams(
            dimension_semantics=("parallel",),
            vmem_limit_bytes=130 * 1024 * 1024),
    )(tablesT)


def _gather_sc(tables_flat, idx):
    num_idx = idx.shape[0]
    d = tables_flat.shape[1]
    idx2 = idx.reshape(1, num_idx)
    mesh = plsc.VectorSubcoreMesh(core_axis_name="c", subcore_axis_name="s")

    @functools.partial(
        pl.kernel,
        out_type=jax.ShapeDtypeStruct((num_idx, d), tables_flat.dtype),
        mesh=mesh,
        compiler_params=pltpu.CompilerParams(use_tc_tiling_on_sc=False),
    )
    def gather_kernel(x_hbm, i_hbm, o_hbm):
        def body(i_vmem, o_vmem):
            pltpu.sync_copy(x_hbm.at[i_vmem.at[0]], o_vmem)

        pltpu.emit_pipeline(
            body,
            grid=(num_idx // _GATHER_WINDOW,),
            in_specs=[pl.BlockSpec((1, _GATHER_WINDOW), lambda i: (0, i))],
            out_specs=[pl.BlockSpec((_GATHER_WINDOW, d), lambda i: (i, 0))],
            core_axis_name=("c", "s"),
            dimension_semantics=(pltpu.PARALLEL,),
        )(i_hbm, o_hbm)

    return gather_kernel(tables_flat, idx2)


def _ln_gelu(h, g, bt):
    mu = jnp.mean(h, axis=-1, keepdims=True)
    var = jnp.mean((h - mu) ** 2, axis=-1, keepdims=True)
    h = (h - mu) / jnp.sqrt(var + 1e-5) * g + bt
    return h * 0.5 * (1.0 + lax.erf(h * (2.0 ** -0.5)))


def _mlp_body(emb_ref, xnum_ref, w1a_ref, w1b_ref, b1_ref, g1_ref, bt1_ref,
              w2_ref, b2_ref, g2_ref, bt2_ref, w3_ref, b3_ref, g3_ref, bt3_ref,
              w4_ref, b4_ref, out_ref):
    dot = functools.partial(jnp.dot, preferred_element_type=jnp.float32,
                            precision=_PREC)
    h = dot(emb_ref[...], w1a_ref[...]) + dot(xnum_ref[...], w1b_ref[...])
    h = _ln_gelu(h + b1_ref[...], g1_ref[...], bt1_ref[...])
    h = _ln_gelu(dot(h, w2_ref[...]) + b2_ref[...], g2_ref[...], bt2_ref[...])
    h = _ln_gelu(dot(h, w3_ref[...]) + b3_ref[...], g3_ref[...], bt3_ref[...])
    z = dot(h, w4_ref[...]) + b4_ref[...]
    out_ref[...] = jax.nn.sigmoid(z)


def _mlp_tc(emb, xnum, w1a, w1b, b1, g1, bt1, w2, b2, g2, bt2,
            w3, b3, g3, bt3, w4, b4):
    bsz = emb.shape[0]
    grid = (bsz // _TB,)

    def tile(r):
        return pl.BlockSpec((_TB, r.shape[1]), lambda i: (i, 0))

    def full(r):
        return pl.BlockSpec(r.shape, lambda i: (0, 0))

    return pl.pallas_call(
        _mlp_body,
        grid=grid,
        in_specs=[tile(emb), tile(xnum)] + [full(r) for r in (
            w1a, w1b, b1, g1, bt1, w2, b2, g2, bt2, w3, b3, g3, bt3, w4, b4)],
        out_specs=pl.BlockSpec((_TB, 1), lambda i: (i, 0)),
        out_shape=jax.ShapeDtypeStruct((bsz, 1), jnp.float32),
        compiler_params=pltpu.CompilerParams(
            dimension_semantics=("parallel",)),
    )(emb, xnum, w1a, w1b, b1, g1, bt1, w2, b2, g2, bt2,
      w3, b3, g3, bt3, w4, b4)


def kernel(x_cat, x_num, tables, W1, b1, g1, bt1, W2, b2, g2, bt2,
           W3, b3, g3, bt3, W4, b4):
    f, v, d = tables.shape
    b = x_cat.shape[0]
    tables_flat = _linearize_tables_tc(
        jnp.swapaxes(tables, 1, 2)).reshape(f * v, d)
    offsets = (jnp.arange(f, dtype=jnp.int32) * v)[None, :]
    idx = (x_cat + offsets).reshape(-1)
    emb = _gather_sc(tables_flat, idx).reshape(b, f * d)
    w1a, w1b = W1[: f * d], W1[f * d:]
    row = lambda x: x.reshape(1, -1)
    return _mlp_tc(emb, x_num, w1a, w1b, row(b1), row(g1), row(bt1),
                   W2, row(b2), row(g2), row(bt2),
                   W3, row(b3), row(g3), row(bt3), W4, row(b4))


# speedup vs baseline: 17.5364x; 1.7279x over previous
import functools

import jax
import jax.numpy as jnp
from jax import lax
from jax.experimental import pallas as pl
from jax.experimental.pallas import tpu as pltpu
from jax.experimental.pallas import tpu_sc as plsc

_GATHER_WINDOW = 128
_TB = 1024
_PACK_CHUNK = 4000
_PREC = lax.Precision.HIGHEST


def _linearize_tables_tc(tablesT):
    f, d, v = tablesT.shape
    rows_per_blk = v * d // 128
    chunk = _PACK_CHUNK
    bounds = list(range(0, v, chunk))

    quarter = chunk // 4

    def body(in_ref, out_ref):
        for c0 in bounds:
            slab = jnp.concatenate(
                [in_ref[0, :, c0 + k * quarter:c0 + (k + 1) * quarter]
                 for k in range(4)], axis=0)
            out_ref[c0 // 4:(c0 + chunk) // 4, :] = slab.T

    return pl.pallas_call(
        body,
        grid=(f,),
        in_specs=[pl.BlockSpec((1, d, v), lambda i: (i, 0, 0))],
        out_specs=pl.BlockSpec((rows_per_blk, 128), lambda i: (i, 0)),
        out_shape=jax.ShapeDtypeStruct((f * v * d // 128, 128), jnp.float32),
        compiler_params=pltpu.CompilerParams(
            dimension_semantics=("parallel",),
            vmem_limit_bytes=130 * 1024 * 1024),
    )(tablesT)


def _gather_sc(tables_flat, idx):
    num_idx = idx.shape[0]
    d = tables_flat.shape[1]
    idx2 = idx.reshape(1, num_idx)
    mesh = plsc.VectorSubcoreMesh(core_axis_name="c", subcore_axis_name="s")

    @functools.partial(
        pl.kernel,
        out_type=jax.ShapeDtypeStruct((num_idx, d), tables_flat.dtype),
        mesh=mesh,
        compiler_params=pltpu.CompilerParams(use_tc_tiling_on_sc=False),
    )
    def gather_kernel(x_hbm, i_hbm, o_hbm):
        def body(i_vmem, o_vmem):
            pltpu.sync_copy(x_hbm.at[i_vmem.at[0]], o_vmem)

        pltpu.emit_pipeline(
            body,
            grid=(num_idx // _GATHER_WINDOW,),
            in_specs=[pl.BlockSpec((1, _GATHER_WINDOW), lambda i: (0, i))],
            out_specs=[pl.BlockSpec((_GATHER_WINDOW, d), lambda i: (i, 0))],
            core_axis_name=("c", "s"),
            dimension_semantics=(pltpu.PARALLEL,),
        )(i_hbm, o_hbm)

    return gather_kernel(tables_flat, idx2)


def _ln_gelu(h, g, bt):
    mu = jnp.mean(h, axis=-1, keepdims=True)
    var = jnp.mean((h - mu) ** 2, axis=-1, keepdims=True)
    h = (h - mu) / jnp.sqrt(var + 1e-5) * g + bt
    return h * 0.5 * (1.0 + lax.erf(h * (2.0 ** -0.5)))


def _mlp_body(emb_ref, xnum_ref, w1a_ref, w1b_ref, b1_ref, g1_ref, bt1_ref,
              w2_ref, b2_ref, g2_ref, bt2_ref, w3_ref, b3_ref, g3_ref, bt3_ref,
              w4_ref, b4_ref, out_ref):
    dot = functools.partial(jnp.dot, preferred_element_type=jnp.float32,
                            precision=_PREC)
    h = dot(emb_ref[...], w1a_ref[...]) + dot(xnum_ref[...], w1b_ref[...])
    h = _ln_gelu(h + b1_ref[...], g1_ref[...], bt1_ref[...])
    h = _ln_gelu(dot(h, w2_ref[...]) + b2_ref[...], g2_ref[...], bt2_ref[...])
    h = _ln_gelu(dot(h, w3_ref[...]) + b3_ref[...], g3_ref[...], bt3_ref[...])
    z = dot(h, w4_ref[...]) + b4_ref[...]
    out_ref[...] = jax.nn.sigmoid(z)


def _mlp_tc(emb, xnum, w1a, w1b, b1, g1, bt1, w2, b2, g2, bt2,
            w3, b3, g3, bt3, w4, b4):
    bsz = emb.shape[0]
    grid = (bsz // _TB,)

    def tile(r):
        return pl.BlockSpec((_TB, r.shape[1]), lambda i: (i, 0))

    def full(r):
        return pl.BlockSpec(r.shape, lambda i: (0, 0))

    return pl.pallas_call(
        _mlp_body,
        grid=grid,
        in_specs=[tile(emb), tile(xnum)] + [full(r) for r in (
            w1a, w1b, b1, g1, bt1, w2, b2, g2, bt2, w3, b3, g3, bt3, w4, b4)],
        out_specs=pl.BlockSpec((_TB, 1), lambda i: (i, 0)),
        out_shape=jax.ShapeDtypeStruct((bsz, 1), jnp.float32),
        compiler_params=pltpu.CompilerParams(
            dimension_semantics=("parallel",)),
    )(emb, xnum, w1a, w1b, b1, g1, bt1, w2, b2, g2, bt2,
      w3, b3, g3, bt3, w4, b4)


def kernel(x_cat, x_num, tables, W1, b1, g1, bt1, W2, b2, g2, bt2,
           W3, b3, g3, bt3, W4, b4):
    f, v, d = tables.shape
    b = x_cat.shape[0]
    tables_flat = _linearize_tables_tc(
        jnp.swapaxes(tables, 1, 2)).reshape(f * v, d)
    offsets = (jnp.arange(f, dtype=jnp.int32) * v)[None, :]
    q = _PACK_CHUNK // 4
    pos = x_cat % _PACK_CHUNK
    perm = (x_cat - pos) + 4 * (pos % q) + pos // q
    idx = (perm + offsets).reshape(-1)
    emb = _gather_sc(tables_flat, idx).reshape(b, f * d)
    w1a, w1b = W1[: f * d], W1[f * d:]
    row = lambda x: x.reshape(1, -1)
    return _mlp_tc(emb, x_num, w1a, w1b, row(b1), row(g1), row(bt1),
                   W2, row(b2), row(g2), row(bt2),
                   W3, row(b3), row(g3), row(bt3), W4, row(b4))


# speedup vs baseline: 21.9154x; 1.2497x over previous
import functools

import jax
import jax.numpy as jnp
from jax import lax
from jax.experimental import pallas as pl
from jax.experimental.pallas import tpu as pltpu
from jax.experimental.pallas import tpu_sc as plsc

_GATHER_WINDOW = 128
_TB = 1024
_PACK_CHUNK = 4000
_PREC = lax.Precision.DEFAULT


def _linearize_tables_tc(tablesT):
    f, d, v = tablesT.shape
    rows_per_blk = v * d // 128
    chunk = _PACK_CHUNK
    bounds = list(range(0, v, chunk))

    quarter = chunk // 4

    def body(in_ref, out_ref):
        for c0 in bounds:
            slab = jnp.concatenate(
                [in_ref[0, :, c0 + k * quarter:c0 + (k + 1) * quarter]
                 for k in range(4)], axis=0)
            out_ref[c0 // 4:(c0 + chunk) // 4, :] = slab.T

    return pl.pallas_call(
        body,
        grid=(f,),
        in_specs=[pl.BlockSpec((1, d, v), lambda i: (i, 0, 0))],
        out_specs=pl.BlockSpec((rows_per_blk, 128), lambda i: (i, 0)),
        out_shape=jax.ShapeDtypeStruct((f * v * d // 128, 128), jnp.float32),
        compiler_params=pltpu.CompilerParams(
            dimension_semantics=("parallel",),
            vmem_limit_bytes=130 * 1024 * 1024),
    )(tablesT)


def _gather_sc(tables_flat, idx):
    num_idx = idx.shape[0]
    d = tables_flat.shape[1]
    idx2 = idx.reshape(1, num_idx)
    mesh = plsc.VectorSubcoreMesh(core_axis_name="c", subcore_axis_name="s")

    @functools.partial(
        pl.kernel,
        out_type=jax.ShapeDtypeStruct((num_idx, d), tables_flat.dtype),
        mesh=mesh,
        compiler_params=pltpu.CompilerParams(use_tc_tiling_on_sc=False),
    )
    def gather_kernel(x_hbm, i_hbm, o_hbm):
        def body(i_vmem, o_vmem):
            pltpu.sync_copy(x_hbm.at[i_vmem.at[0]], o_vmem)

        pltpu.emit_pipeline(
            body,
            grid=(num_idx // _GATHER_WINDOW,),
            in_specs=[pl.BlockSpec((1, _GATHER_WINDOW), lambda i: (0, i))],
            out_specs=[pl.BlockSpec((_GATHER_WINDOW, d), lambda i: (i, 0))],
            core_axis_name=("c", "s"),
            dimension_semantics=(pltpu.PARALLEL,),
        )(i_hbm, o_hbm)

    return gather_kernel(tables_flat, idx2)


def _ln_gelu(h, g, bt):
    mu = jnp.mean(h, axis=-1, keepdims=True)
    var = jnp.mean((h - mu) ** 2, axis=-1, keepdims=True)
    h = (h - mu) / jnp.sqrt(var + 1e-5) * g + bt
    return h * 0.5 * (1.0 + lax.erf(h * (2.0 ** -0.5)))


def _mlp_body(emb_ref, xnum_ref, w1a_ref, w1b_ref, b1_ref, g1_ref, bt1_ref,
              w2_ref, b2_ref, g2_ref, bt2_ref, w3_ref, b3_ref, g3_ref, bt3_ref,
              w4_ref, b4_ref, out_ref):
    dot = functools.partial(jnp.dot, preferred_element_type=jnp.float32,
                            precision=_PREC)
    h = dot(emb_ref[...], w1a_ref[...]) + dot(xnum_ref[...], w1b_ref[...])
    h = _ln_gelu(h + b1_ref[...], g1_ref[...], bt1_ref[...])
    h = _ln_gelu(dot(h, w2_ref[...]) + b2_ref[...], g2_ref[...], bt2_ref[...])
    h = _ln_gelu(dot(h, w3_ref[...]) + b3_ref[...], g3_ref[...], bt3_ref[...])
    z = dot(h, w4_ref[...]) + b4_ref[...]
    out_ref[...] = jax.nn.sigmoid(z)


def _mlp_tc(emb, xnum, w1a, w1b, b1, g1, bt1, w2, b2, g2, bt2,
            w3, b3, g3, bt3, w4, b4):
    bsz = emb.shape[0]
    grid = (bsz // _TB,)

    def tile(r):
        return pl.BlockSpec((_TB, r.shape[1]), lambda i: (i, 0))

    def full(r):
        return pl.BlockSpec(r.shape, lambda i: (0, 0))

    return pl.pallas_call(
        _mlp_body,
        grid=grid,
        in_specs=[tile(emb), tile(xnum)] + [full(r) for r in (
            w1a, w1b, b1, g1, bt1, w2, b2, g2, bt2, w3, b3, g3, bt3, w4, b4)],
        out_specs=pl.BlockSpec((_TB, 1), lambda i: (i, 0)),
        out_shape=jax.ShapeDtypeStruct((bsz, 1), jnp.float32),
        compiler_params=pltpu.CompilerParams(
            dimension_semantics=("parallel",)),
    )(emb, xnum, w1a, w1b, b1, g1, bt1, w2, b2, g2, bt2,
      w3, b3, g3, bt3, w4, b4)


def kernel(x_cat, x_num, tables, W1, b1, g1, bt1, W2, b2, g2, bt2,
           W3, b3, g3, bt3, W4, b4):
    f, v, d = tables.shape
    b = x_cat.shape[0]
    tables_flat = _linearize_tables_tc(
        jnp.swapaxes(tables, 1, 2)).reshape(f * v, d)
    offsets = (jnp.arange(f, dtype=jnp.int32) * v)[None, :]
    q = _PACK_CHUNK // 4
    pos = x_cat % _PACK_CHUNK
    perm = (x_cat - pos) + 4 * (pos % q) + pos // q
    idx = (perm + offsets).reshape(-1)
    emb = _gather_sc(tables_flat, idx).reshape(b, f * d)
    w1a, w1b = W1[: f * d], W1[f * d:]
    row = lambda x: x.reshape(1, -1)
    return _mlp_tc(emb, x_num, w1a, w1b, row(b1), row(g1), row(bt1),
                   W2, row(b2), row(g2), row(bt2),
                   W3, row(b3), row(g3), row(bt3), W4, row(b4))
